# Initial kernel scaffold; baseline (speedup 1.0000x reference)
import jax
import jax.numpy as jnp
from jax.experimental import pallas as pl


def kernel(x, edge_index, W1l, b1l, W1r, W2l, b2l, W2r, Wm1, bm1, Wm2, bm2):
    raise NotImplementedError("write your pallas kernel here")


# speedup vs baseline: 5.8203x; 5.8203x over previous
import functools

import jax
import jax.numpy as jnp
from jax import lax
from jax.experimental import pallas as pl
from jax.experimental.pallas import tpu as pltpu
from jax.experimental.pallas import tpu_sc as plsc

NN = 10000
DD = 128
CC = 16
EE = 320000

NC = 2
NS = 16
K = 80
CH = EE // (NC * NS * K)
G = 5
NG = CH // G
NP = 10240
RPT = NP // NS


def _dot(a, b):
    return lax.dot_general(a, b, (((1,), (0,)), ((), ())),
                           precision=lax.Precision.HIGHEST,
                           preferred_element_type=jnp.float32)


def _elu(x):
    return jnp.where(x > 0, x, jnp.exp(jnp.minimum(x, 0.0)) - 1.0)


mesh_v = plsc.VectorSubcoreMesh(core_axis_name="c", subcore_axis_name="s")


def _zero_rows(rows_v, val=0.0):
    vv = jnp.full((16,), val, jnp.float32)

    def zrow(i, carry):
        for j in range(DD // 16):
            rows_v[i, pl.ds(j * 16, 16)] = vv
        return carry

    lax.fori_loop(0, K, zrow, 0)


def _make_sc_agg():
    def body(p_hbm, src_hbm, dst_hbm, agg_out,
             src_v, dst_v, rows_v, sem, acc_sh):
        c = lax.axis_index("c")
        s = lax.axis_index("s")
        base = s * RPT
        _zero_rows(rows_v)

        def zcp(ii, carry):
            pltpu.sync_copy(rows_v, acc_sh.at[pl.ds(base + ii * K, K)])
            return carry

        lax.fori_loop(0, RPT // K, zcp, 0)
        plsc.subcore_barrier()

        def group(g, carry):
            pltpu.sync_copy(src_hbm.at[c, s, g], src_v)
            pltpu.sync_copy(dst_hbm.at[c, s, g], dst_v)
            for jj in range(G):
                pltpu.async_copy(p_hbm.at[src_v.at[jj]], rows_v, sem).wait()
                pltpu.sync_copy(rows_v, acc_sh.at[dst_v.at[jj]], add=True)
            return carry

        lax.fori_loop(0, NG, group, 0)
        plsc.subcore_barrier()

        def drain(ii, carry):
            pltpu.sync_copy(acc_sh.at[pl.ds(base + ii * K, K)], rows_v)
            pltpu.sync_copy(rows_v, agg_out.at[c, pl.ds(base + ii * K, K)])
            return carry

        lax.fori_loop(0, RPT // K, drain, 0)

    return pl.kernel(
        body,
        out_type=jax.ShapeDtypeStruct((NC, NP, DD), jnp.float32),
        mesh=mesh_v,
        scratch_types=[
            pltpu.VMEM((G, K), jnp.int32),
            pltpu.VMEM((G, K), jnp.int32),
            pltpu.VMEM((K, DD), jnp.float32),
            pltpu.SemaphoreType.DMA,
            pltpu.VMEM_SHARED((NP, DD), jnp.float32),
        ])


def _make_sc_deg():
    def body(dst_hbm, deg_out, dst_v, rows_v, acc_sh):
        c = lax.axis_index("c")
        s = lax.axis_index("s")
        base = s * RPT
        _zero_rows(rows_v)

        def zcp(ii, carry):
            pltpu.sync_copy(rows_v, acc_sh.at[pl.ds(base + ii * K, K)])
            return carry

        lax.fori_loop(0, RPT // K, zcp, 0)
        plsc.subcore_barrier()
        _zero_rows(rows_v, 1.0)

        def group(g, carry):
            pltpu.sync_copy(dst_hbm.at[c, s, g], dst_v)
            for jj in range(G):
                pltpu.sync_copy(rows_v, acc_sh.at[dst_v.at[jj]], add=True)
            return carry

        lax.fori_loop(0, NG, group, 0)
        plsc.subcore_barrier()

        def drain(ii, carry):
            pltpu.sync_copy(acc_sh.at[pl.ds(base + ii * K, K)], rows_v)
            pltpu.sync_copy(rows_v, deg_out.at[c, pl.ds(base + ii * K, K)])
            return carry

        lax.fori_loop(0, RPT // K, drain, 0)

    return pl.kernel(
        body,
        out_type=jax.ShapeDtypeStruct((NC, NP, DD), jnp.float32),
        mesh=mesh_v,
        scratch_types=[
            pltpu.VMEM((G, K), jnp.int32),
            pltpu.VMEM((K, DD), jnp.float32),
            pltpu.VMEM_SHARED((NP, DD), jnp.float32),
        ])


_TB = 1000


def _tc1(x, W1l, W1r, b1l):
    def body(x_ref, wl_ref, wr_ref, b_ref, p_ref, r_ref):
        xb = x_ref[...]
        p_ref[...] = _dot(xb, wl_ref[...])
        r_ref[...] = _dot(xb, wr_ref[...]) + b_ref[...]

    return pl.pallas_call(
        body,
        grid=(NN // _TB,),
        in_specs=[
            pl.BlockSpec((_TB, DD), lambda i: (i, 0)),
            pl.BlockSpec((DD, DD), lambda i: (0, 0)),
            pl.BlockSpec((DD, DD), lambda i: (0, 0)),
            pl.BlockSpec((1, DD), lambda i: (0, 0)),
        ],
        out_specs=[pl.BlockSpec((_TB, DD), lambda i: (i, 0))] * 2,
        out_shape=[jax.ShapeDtypeStruct((NN, DD), jnp.float32)] * 2,
    )(x, W1l, W1r, b1l.reshape(1, DD))


def _combine(aggp_ref, degp_ref, r_ref):
    agg = aggp_ref[0] + aggp_ref[1]
    deg = degp_ref[0, :, 0:1] + degp_ref[1, :, 0:1]
    deg = jnp.maximum(deg, 1.0)
    return _elu(agg / deg + r_ref[...])


def _tc2(aggp, degp, R1, W2l, W2r, b2l):
    def body(aggp_ref, degp_ref, r1_ref, wl_ref, wr_ref, b_ref,
             p_ref, r_ref):
        h = _combine(aggp_ref, degp_ref, r1_ref)
        p_ref[...] = _dot(h, wl_ref[...])
        r_ref[...] = _dot(h, wr_ref[...]) + b_ref[...]

    return pl.pallas_call(
        body,
        grid=(NN // _TB,),
        in_specs=[
            pl.BlockSpec((NC, _TB, DD), lambda i: (0, i, 0)),
            pl.BlockSpec((NC, _TB, DD), lambda i: (0, i, 0)),
            pl.BlockSpec((_TB, DD), lambda i: (i, 0)),
            pl.BlockSpec((DD, DD), lambda i: (0, 0)),
            pl.BlockSpec((DD, DD), lambda i: (0, 0)),
            pl.BlockSpec((1, DD), lambda i: (0, 0)),
        ],
        out_specs=[pl.BlockSpec((_TB, DD), lambda i: (i, 0))] * 2,
        out_shape=[jax.ShapeDtypeStruct((NN, DD), jnp.float32)] * 2,
    )(aggp, degp, R1, W2l, W2r, b2l.reshape(1, DD))


def _tc3(aggp, degp, R2, Wm1, bm1, Wm2, bm2):
    def body(aggp_ref, degp_ref, r2_ref, w1_ref, b1_ref, w2_ref, b2_ref,
             o_ref):
        h = _combine(aggp_ref, degp_ref, r2_ref)
        t = jnp.maximum(_dot(h, w1_ref[...]) + b1_ref[...], 0.0)
        o_ref[...] = jnp.maximum(_dot(t, w2_ref[...]) + b2_ref[...], 0.0)

    return pl.pallas_call(
        body,
        grid=(NN // _TB,),
        in_specs=[
            pl.BlockSpec((NC, _TB, DD), lambda i: (0, i, 0)),
            pl.BlockSpec((NC, _TB, DD), lambda i: (0, i, 0)),
            pl.BlockSpec((_TB, DD), lambda i: (i, 0)),
            pl.BlockSpec((DD, DD), lambda i: (0, 0)),
            pl.BlockSpec((1, DD), lambda i: (0, 0)),
            pl.BlockSpec((DD, CC), lambda i: (0, 0)),
            pl.BlockSpec((1, CC), lambda i: (0, 0)),
        ],
        out_specs=pl.BlockSpec((_TB, CC), lambda i: (i, 0)),
        out_shape=jax.ShapeDtypeStruct((NN, CC), jnp.float32),
    )(aggp, degp, R2, Wm1, bm1.reshape(1, DD), Wm2, bm2.reshape(1, CC))


def kernel(x, edge_index, W1l, b1l, W1r, W2l, b2l, W2r, Wm1, bm1, Wm2, bm2):
    src4 = edge_index[0].reshape(NC, NS, NG, G, K)
    dst4 = edge_index[1].reshape(NC, NS, NG, G, K)

    sc_agg = _make_sc_agg()
    sc_deg = _make_sc_deg()

    degp = sc_deg(dst4)
    P1, R1 = _tc1(x, W1l, W1r, b1l)
    aggp1 = sc_agg(P1, src4, dst4)
    P2, R2 = _tc2(aggp1, degp, R1, W2l, W2r, b2l)
    aggp2 = sc_agg(P2, src4, dst4)
    return _tc3(aggp2, degp, R2, Wm1, bm1, Wm2, bm2)


# speedup vs baseline: 8.2231x; 1.4128x over previous
import functools

import jax
import jax.numpy as jnp
from jax import lax
from jax.experimental import pallas as pl
from jax.experimental.pallas import tpu as pltpu
from jax.experimental.pallas import tpu_sc as plsc

NN = 10000
DD = 128
CC = 16
EE = 320000

NC = 2
NS = 16
K = 80
CH = EE // (NC * NS * K)
G = 5
NG = CH // G
NP = 10240
RPT = NP // NS


def _dot(a, b):
    return lax.dot_general(a, b, (((1,), (0,)), ((), ())),
                           precision=lax.Precision.HIGHEST,
                           preferred_element_type=jnp.float32)


def _elu(x):
    return jnp.where(x > 0, x, jnp.exp(jnp.minimum(x, 0.0)) - 1.0)


mesh_v = plsc.VectorSubcoreMesh(core_axis_name="c", subcore_axis_name="s")


def _zero_rows(rows_v, val=0.0):
    vv = jnp.full((16,), val, jnp.float32)

    def zrow(i, carry):
        for j in range(DD // 16):
            rows_v[i, pl.ds(j * 16, 16)] = vv
        return carry

    lax.fori_loop(0, K, zrow, 0)


def _make_sc_agg():
    def body(p_hbm, src_hbm, dst_hbm, agg_out,
             src_v, dst_v, rows_v, gsem, isem, acc_sh):
        c = lax.axis_index("c")
        s = lax.axis_index("s")
        base = s * RPT
        _zero_rows(rows_v.at[0])

        def zcp(ii, carry):
            pltpu.sync_copy(rows_v.at[0], acc_sh.at[pl.ds(base + ii * K, K)])
            return carry

        lax.fori_loop(0, RPT // K, zcp, 0)
        plsc.subcore_barrier()

        pltpu.sync_copy(src_hbm.at[c, s, 0], src_v.at[0])
        pltpu.sync_copy(dst_hbm.at[c, s, 0], dst_v.at[0])

        def group(g, carry):
            b = lax.rem(g, 2)
            nb = lax.rem(g + 1, 2)

            @pl.when(g + 1 < NG)
            def _():
                pltpu.async_copy(src_hbm.at[c, s, g + 1], src_v.at[nb],
                                 isem)
                pltpu.async_copy(dst_hbm.at[c, s, g + 1], dst_v.at[nb],
                                 isem)

            pltpu.async_copy(p_hbm.at[src_v.at[b, 0]], rows_v.at[0], gsem)
            for jj in range(G):
                if jj + 1 < G:
                    pltpu.async_copy(p_hbm.at[src_v.at[b, jj + 1]],
                                     rows_v.at[(jj + 1) % 2], gsem)
                pltpu.make_async_copy(p_hbm.at[src_v.at[b, jj]],
                                      rows_v.at[jj % 2], gsem).wait()
                pltpu.sync_copy(rows_v.at[jj % 2],
                                acc_sh.at[dst_v.at[b, jj]], add=True)

            @pl.when(g + 1 < NG)
            def _():
                pltpu.make_async_copy(src_hbm.at[c, s, 0], src_v.at[nb],
                                      isem).wait()
                pltpu.make_async_copy(dst_hbm.at[c, s, 0], dst_v.at[nb],
                                      isem).wait()

            return carry

        lax.fori_loop(0, NG, group, 0)
        plsc.subcore_barrier()

        def drain(ii, carry):
            pltpu.sync_copy(acc_sh.at[pl.ds(base + ii * K, K)],
                            rows_v.at[0])
            pltpu.sync_copy(rows_v.at[0],
                            agg_out.at[c, pl.ds(base + ii * K, K)])
            return carry

        lax.fori_loop(0, RPT // K, drain, 0)

    return pl.kernel(
        body,
        out_type=jax.ShapeDtypeStruct((NC, NP, DD), jnp.float32),
        mesh=mesh_v,
        scratch_types=[
            pltpu.VMEM((2, G, K), jnp.int32),
            pltpu.VMEM((2, G, K), jnp.int32),
            pltpu.VMEM((2, K, DD), jnp.float32),
            pltpu.SemaphoreType.DMA,
            pltpu.SemaphoreType.DMA,
            pltpu.VMEM_SHARED((NP, DD), jnp.float32),
        ])


def _make_sc_deg():
    def body(dst_hbm, deg_out, dst_v, rows_v, acc_sh):
        c = lax.axis_index("c")
        s = lax.axis_index("s")
        base = s * RPT
        _zero_rows(rows_v)

        def zcp(ii, carry):
            pltpu.sync_copy(rows_v, acc_sh.at[pl.ds(base + ii * K, K)])
            return carry

        lax.fori_loop(0, RPT // K, zcp, 0)
        plsc.subcore_barrier()
        _zero_rows(rows_v, 1.0)

        def group(g, carry):
            pltpu.sync_copy(dst_hbm.at[c, s, g], dst_v)
            for jj in range(G):
                pltpu.sync_copy(rows_v, acc_sh.at[dst_v.at[jj]], add=True)
            return carry

        lax.fori_loop(0, NG, group, 0)
        plsc.subcore_barrier()

        def drain(ii, carry):
            pltpu.sync_copy(acc_sh.at[pl.ds(base + ii * K, K)], rows_v)
            pltpu.sync_copy(rows_v, deg_out.at[c, pl.ds(base + ii * K, K)])
            return carry

        lax.fori_loop(0, RPT // K, drain, 0)

    return pl.kernel(
        body,
        out_type=jax.ShapeDtypeStruct((NC, NP, DD), jnp.float32),
        mesh=mesh_v,
        scratch_types=[
            pltpu.VMEM((G, K), jnp.int32),
            pltpu.VMEM((K, DD), jnp.float32),
            pltpu.VMEM_SHARED((NP, DD), jnp.float32),
        ])


_TB = 1000


def _tc1(x, W1l, W1r, b1l):
    def body(x_ref, wl_ref, wr_ref, b_ref, p_ref, r_ref):
        xb = x_ref[...]
        p_ref[...] = _dot(xb, wl_ref[...])
        r_ref[...] = _dot(xb, wr_ref[...]) + b_ref[...]

    return pl.pallas_call(
        body,
        grid=(NN // _TB,),
        in_specs=[
            pl.BlockSpec((_TB, DD), lambda i: (i, 0)),
            pl.BlockSpec((DD, DD), lambda i: (0, 0)),
            pl.BlockSpec((DD, DD), lambda i: (0, 0)),
            pl.BlockSpec((1, DD), lambda i: (0, 0)),
        ],
        out_specs=[pl.BlockSpec((_TB, DD), lambda i: (i, 0))] * 2,
        out_shape=[jax.ShapeDtypeStruct((NN, DD), jnp.float32)] * 2,
    )(x, W1l, W1r, b1l.reshape(1, DD))


def _combine(aggp_ref, degp_ref, r_ref):
    agg = aggp_ref[0] + aggp_ref[1]
    deg = degp_ref[0, :, 0:1] + degp_ref[1, :, 0:1]
    deg = jnp.maximum(deg, 1.0)
    return _elu(agg / deg + r_ref[...])


def _tc2(aggp, degp, R1, W2l, W2r, b2l):
    def body(aggp_ref, degp_ref, r1_ref, wl_ref, wr_ref, b_ref,
             p_ref, r_ref):
        h = _combine(aggp_ref, degp_ref, r1_ref)
        p_ref[...] = _dot(h, wl_ref[...])
        r_ref[...] = _dot(h, wr_ref[...]) + b_ref[...]

    return pl.pallas_call(
        body,
        grid=(NN // _TB,),
        in_specs=[
            pl.BlockSpec((NC, _TB, DD), lambda i: (0, i, 0)),
            pl.BlockSpec((NC, _TB, DD), lambda i: (0, i, 0)),
            pl.BlockSpec((_TB, DD), lambda i: (i, 0)),
            pl.BlockSpec((DD, DD), lambda i: (0, 0)),
            pl.BlockSpec((DD, DD), lambda i: (0, 0)),
            pl.BlockSpec((1, DD), lambda i: (0, 0)),
        ],
        out_specs=[pl.BlockSpec((_TB, DD), lambda i: (i, 0))] * 2,
        out_shape=[jax.ShapeDtypeStruct((NN, DD), jnp.float32)] * 2,
    )(aggp, degp, R1, W2l, W2r, b2l.reshape(1, DD))


def _tc3(aggp, degp, R2, Wm1, bm1, Wm2, bm2):
    def body(aggp_ref, degp_ref, r2_ref, w1_ref, b1_ref, w2_ref, b2_ref,
             o_ref):
        h = _combine(aggp_ref, degp_ref, r2_ref)
        t = jnp.maximum(_dot(h, w1_ref[...]) + b1_ref[...], 0.0)
        o_ref[...] = jnp.maximum(_dot(t, w2_ref[...]) + b2_ref[...], 0.0)

    return pl.pallas_call(
        body,
        grid=(NN // _TB,),
        in_specs=[
            pl.BlockSpec((NC, _TB, DD), lambda i: (0, i, 0)),
            pl.BlockSpec((NC, _TB, DD), lambda i: (0, i, 0)),
            pl.BlockSpec((_TB, DD), lambda i: (i, 0)),
            pl.BlockSpec((DD, DD), lambda i: (0, 0)),
            pl.BlockSpec((1, DD), lambda i: (0, 0)),
            pl.BlockSpec((DD, CC), lambda i: (0, 0)),
            pl.BlockSpec((1, CC), lambda i: (0, 0)),
        ],
        out_specs=pl.BlockSpec((_TB, CC), lambda i: (i, 0)),
        out_shape=jax.ShapeDtypeStruct((NN, CC), jnp.float32),
    )(aggp, degp, R2, Wm1, bm1.reshape(1, DD), Wm2, bm2.reshape(1, CC))


def kernel(x, edge_index, W1l, b1l, W1r, W2l, b2l, W2r, Wm1, bm1, Wm2, bm2):
    src4 = edge_index[0].reshape(NC, NS, NG, G, K)
    dst4 = edge_index[1].reshape(NC, NS, NG, G, K)

    sc_agg = _make_sc_agg()
    sc_deg = _make_sc_deg()

    degp = sc_deg(dst4)
    P1, R1 = _tc1(x, W1l, W1r, b1l)
    aggp1 = sc_agg(P1, src4, dst4)
    P2, R2 = _tc2(aggp1, degp, R1, W2l, W2r, b2l)
    aggp2 = sc_agg(P2, src4, dst4)
    return _tc3(aggp2, degp, R2, Wm1, bm1, Wm2, bm2)


# speedup vs baseline: 9.7094x; 1.1807x over previous
import functools

import jax
import jax.numpy as jnp
from jax import lax
from jax.experimental import pallas as pl
from jax.experimental.pallas import tpu as pltpu
from jax.experimental.pallas import tpu_sc as plsc

NN = 10000
DD = 128
CC = 16
EE = 320000

NC = 2
NS = 16
K = 80
CH = EE // (NC * NS * K)
G = 5
NG = CH // G
NP = 10240
RPT = NP // NS


def _dot(a, b):
    return lax.dot_general(a, b, (((1,), (0,)), ((), ())),
                           precision=lax.Precision.HIGHEST,
                           preferred_element_type=jnp.float32)


def _elu(x):
    return jnp.where(x > 0, x, jnp.exp(jnp.minimum(x, 0.0)) - 1.0)


mesh_v = plsc.VectorSubcoreMesh(core_axis_name="c", subcore_axis_name="s")


def _zero_rows(rows_v, val=0.0):
    vv = jnp.full((16,), val, jnp.float32)

    def zrow(i, carry):
        for j in range(DD // 16):
            rows_v[i, pl.ds(j * 16, 16)] = vv
        return carry

    lax.fori_loop(0, K, zrow, 0)


def _make_sc_agg():
    def body(p_hbm, src_hbm, dst_hbm, agg_out,
             src_v, dst_v, rows_v, gsem, isem, ssem, acc_sh):
        c = lax.axis_index("c")
        s = lax.axis_index("s")
        base = s * RPT
        _zero_rows(rows_v.at[0])

        def zcp(ii, carry):
            pltpu.sync_copy(rows_v.at[0], acc_sh.at[pl.ds(base + ii * K, K)])
            return carry

        lax.fori_loop(0, RPT // K, zcp, 0)
        plsc.subcore_barrier()

        pltpu.sync_copy(src_hbm.at[c, s, 0], src_v.at[0])
        pltpu.sync_copy(dst_hbm.at[c, s, 0], dst_v.at[0])
        pltpu.async_copy(p_hbm.at[src_v.at[0, 0]], rows_v.at[0], gsem)

        def wait_one(ref_slices, sem):
            pltpu.make_async_copy(*ref_slices, sem).wait()

        def group(g, carry):
            b = lax.rem(g, 2)
            nb = lax.rem(g + 1, 2)

            @pl.when(g + 1 < NG)
            def _():
                pltpu.async_copy(src_hbm.at[c, s, g + 1], src_v.at[nb],
                                 isem)
                pltpu.async_copy(dst_hbm.at[c, s, g + 1], dst_v.at[nb],
                                 isem)

            for jj in range(G):
                t = g * G + jj
                bt = lax.rem(t, 3)
                bt1 = lax.rem(t + 1, 3)

                @pl.when(t >= 2)
                def _():
                    wait_one((p_hbm.at[src_v.at[b, 0]], rows_v.at[0]),
                             ssem)

                if jj + 1 < G:
                    pltpu.async_copy(p_hbm.at[src_v.at[b, jj + 1]],
                                     rows_v.at[bt1], gsem)
                else:
                    @pl.when(g + 1 < NG)
                    def _():
                        wait_one((src_hbm.at[c, s, 0], src_v.at[nb]),
                                 isem)
                        wait_one((dst_hbm.at[c, s, 0], dst_v.at[nb]),
                                 isem)
                        pltpu.async_copy(p_hbm.at[src_v.at[nb, 0]],
                                         rows_v.at[bt1], gsem)

                wait_one((p_hbm.at[src_v.at[b, jj]], rows_v.at[bt]), gsem)
                pltpu.async_copy(rows_v.at[bt],
                                 acc_sh.at[dst_v.at[b, jj]], ssem,
                                 add=True)
            return carry

        lax.fori_loop(0, NG, group, 0)
        wait_one((p_hbm.at[src_v.at[0, 0]], rows_v.at[0]), ssem)
        wait_one((p_hbm.at[src_v.at[0, 0]], rows_v.at[0]), ssem)
        plsc.subcore_barrier()

        def drain(ii, carry):
            pltpu.sync_copy(acc_sh.at[pl.ds(base + ii * K, K)],
                            rows_v.at[0])
            pltpu.sync_copy(rows_v.at[0],
                            agg_out.at[c, pl.ds(base + ii * K, K)])
            return carry

        lax.fori_loop(0, RPT // K, drain, 0)

    return pl.kernel(
        body,
        out_type=jax.ShapeDtypeStruct((NC, NP, DD), jnp.float32),
        mesh=mesh_v,
        scratch_types=[
            pltpu.VMEM((2, G, K), jnp.int32),
            pltpu.VMEM((2, G, K), jnp.int32),
            pltpu.VMEM((3, K, DD), jnp.float32),
            pltpu.SemaphoreType.DMA,
            pltpu.SemaphoreType.DMA,
            pltpu.SemaphoreType.DMA,
            pltpu.VMEM_SHARED((NP, DD), jnp.float32),
        ])


def _make_sc_deg():
    def body(dst_hbm, deg_out, dst_v, rows_v, acc_sh):
        c = lax.axis_index("c")
        s = lax.axis_index("s")
        base = s * RPT
        _zero_rows(rows_v)

        def zcp(ii, carry):
            pltpu.sync_copy(rows_v, acc_sh.at[pl.ds(base + ii * K, K)])
            return carry

        lax.fori_loop(0, RPT // K, zcp, 0)
        plsc.subcore_barrier()
        _zero_rows(rows_v, 1.0)

        def group(g, carry):
            pltpu.sync_copy(dst_hbm.at[c, s, g], dst_v)
            for jj in range(G):
                pltpu.sync_copy(rows_v, acc_sh.at[dst_v.at[jj]], add=True)
            return carry

        lax.fori_loop(0, NG, group, 0)
        plsc.subcore_barrier()

        def drain(ii, carry):
            pltpu.sync_copy(acc_sh.at[pl.ds(base + ii * K, K)], rows_v)
            pltpu.sync_copy(rows_v, deg_out.at[c, pl.ds(base + ii * K, K)])
            return carry

        lax.fori_loop(0, RPT // K, drain, 0)

    return pl.kernel(
        body,
        out_type=jax.ShapeDtypeStruct((NC, NP, DD), jnp.float32),
        mesh=mesh_v,
        scratch_types=[
            pltpu.VMEM((G, K), jnp.int32),
            pltpu.VMEM((K, DD), jnp.float32),
            pltpu.VMEM_SHARED((NP, DD), jnp.float32),
        ])


_TB = 1000


def _tc1(x, W1l, W1r, b1l):
    def body(x_ref, wl_ref, wr_ref, b_ref, p_ref, r_ref):
        xb = x_ref[...]
        p_ref[...] = _dot(xb, wl_ref[...])
        r_ref[...] = _dot(xb, wr_ref[...]) + b_ref[...]

    return pl.pallas_call(
        body,
        grid=(NN // _TB,),
        in_specs=[
            pl.BlockSpec((_TB, DD), lambda i: (i, 0)),
            pl.BlockSpec((DD, DD), lambda i: (0, 0)),
            pl.BlockSpec((DD, DD), lambda i: (0, 0)),
            pl.BlockSpec((1, DD), lambda i: (0, 0)),
        ],
        out_specs=[pl.BlockSpec((_TB, DD), lambda i: (i, 0))] * 2,
        out_shape=[jax.ShapeDtypeStruct((NN, DD), jnp.float32)] * 2,
    )(x, W1l, W1r, b1l.reshape(1, DD))


def _combine(aggp_ref, degp_ref, r_ref):
    agg = aggp_ref[0] + aggp_ref[1]
    deg = degp_ref[0, :, 0:1] + degp_ref[1, :, 0:1]
    deg = jnp.maximum(deg, 1.0)
    return _elu(agg / deg + r_ref[...])


def _tc2(aggp, degp, R1, W2l, W2r, b2l):
    def body(aggp_ref, degp_ref, r1_ref, wl_ref, wr_ref, b_ref,
             p_ref, r_ref):
        h = _combine(aggp_ref, degp_ref, r1_ref)
        p_ref[...] = _dot(h, wl_ref[...])
        r_ref[...] = _dot(h, wr_ref[...]) + b_ref[...]

    return pl.pallas_call(
        body,
        grid=(NN // _TB,),
        in_specs=[
            pl.BlockSpec((NC, _TB, DD), lambda i: (0, i, 0)),
            pl.BlockSpec((NC, _TB, DD), lambda i: (0, i, 0)),
            pl.BlockSpec((_TB, DD), lambda i: (i, 0)),
            pl.BlockSpec((DD, DD), lambda i: (0, 0)),
            pl.BlockSpec((DD, DD), lambda i: (0, 0)),
            pl.BlockSpec((1, DD), lambda i: (0, 0)),
        ],
        out_specs=[pl.BlockSpec((_TB, DD), lambda i: (i, 0))] * 2,
        out_shape=[jax.ShapeDtypeStruct((NN, DD), jnp.float32)] * 2,
    )(aggp, degp, R1, W2l, W2r, b2l.reshape(1, DD))


def _tc3(aggp, degp, R2, Wm1, bm1, Wm2, bm2):
    def body(aggp_ref, degp_ref, r2_ref, w1_ref, b1_ref, w2_ref, b2_ref,
             o_ref):
        h = _combine(aggp_ref, degp_ref, r2_ref)
        t = jnp.maximum(_dot(h, w1_ref[...]) + b1_ref[...], 0.0)
        o_ref[...] = jnp.maximum(_dot(t, w2_ref[...]) + b2_ref[...], 0.0)

    return pl.pallas_call(
        body,
        grid=(NN // _TB,),
        in_specs=[
            pl.BlockSpec((NC, _TB, DD), lambda i: (0, i, 0)),
            pl.BlockSpec((NC, _TB, DD), lambda i: (0, i, 0)),
            pl.BlockSpec((_TB, DD), lambda i: (i, 0)),
            pl.BlockSpec((DD, DD), lambda i: (0, 0)),
            pl.BlockSpec((1, DD), lambda i: (0, 0)),
            pl.BlockSpec((DD, CC), lambda i: (0, 0)),
            pl.BlockSpec((1, CC), lambda i: (0, 0)),
        ],
        out_specs=pl.BlockSpec((_TB, CC), lambda i: (i, 0)),
        out_shape=jax.ShapeDtypeStruct((NN, CC), jnp.float32),
    )(aggp, degp, R2, Wm1, bm1.reshape(1, DD), Wm2, bm2.reshape(1, CC))


def kernel(x, edge_index, W1l, b1l, W1r, W2l, b2l, W2r, Wm1, bm1, Wm2, bm2):
    src4 = edge_index[0].reshape(NC, NS, NG, G, K)
    dst4 = edge_index[1].reshape(NC, NS, NG, G, K)

    sc_agg = _make_sc_agg()
    sc_deg = _make_sc_deg()

    degp = sc_deg(dst4)
    P1, R1 = _tc1(x, W1l, W1r, b1l)
    aggp1 = sc_agg(P1, src4, dst4)
    P2, R2 = _tc2(aggp1, degp, R1, W2l, W2r, b2l)
    aggp2 = sc_agg(P2, src4, dst4)
    return _tc3(aggp2, degp, R2, Wm1, bm1, Wm2, bm2)


# speedup vs baseline: 9.9583x; 1.0256x over previous
import functools

import jax
import jax.numpy as jnp
from jax import lax
from jax.experimental import pallas as pl
from jax.experimental.pallas import tpu as pltpu
from jax.experimental.pallas import tpu_sc as plsc

NN = 10000
DD = 128
CC = 16
EE = 320000

NC = 2
NS = 16
K = 80
CH = EE // (NC * NS * K)
G = 5
NG = CH // G
NP = 10240
RPT = NP // NS


def _dot(a, b):
    return lax.dot_general(a, b, (((1,), (0,)), ((), ())),
                           precision=lax.Precision.HIGHEST,
                           preferred_element_type=jnp.float32)


def _elu(x):
    return jnp.where(x > 0, x, jnp.exp(jnp.minimum(x, 0.0)) - 1.0)


mesh_v = plsc.VectorSubcoreMesh(core_axis_name="c", subcore_axis_name="s")


def _zero_rows(rows_v, val=0.0):
    vv = jnp.full((16,), val, jnp.float32)

    def zrow(i, carry):
        for j in range(DD // 16):
            rows_v[i, pl.ds(j * 16, 16)] = vv
        return carry

    lax.fori_loop(0, K, zrow, 0)


def _zero_phase(rows0, acc_sh, base, sem):
    for ii in range(RPT // K):
        pltpu.async_copy(rows0, acc_sh.at[pl.ds(base + ii * K, K)], sem)
    for ii in range(RPT // K):
        pltpu.make_async_copy(rows0,
                              acc_sh.at[pl.ds(base, K)], sem).wait()


def _drain_phase(acc_sh, out_hbm, c, base, rows_v, insem, outsem):
    n = RPT // K

    def acc_at(ii):
        return acc_sh.at[pl.ds(base + ii * K, K)]

    def out_at(ii):
        return out_hbm.at[c, pl.ds(base + ii * K, K)]

    pltpu.async_copy(acc_at(0), rows_v.at[0], insem)
    nwait = 0
    for ii in range(n):
        if ii + 1 < n:
            if ii >= 2:
                pltpu.make_async_copy(rows_v.at[0], out_at(0),
                                      outsem).wait()
                nwait += 1
            pltpu.async_copy(acc_at(ii + 1), rows_v.at[(ii + 1) % 3],
                             insem)
        pltpu.make_async_copy(acc_at(ii), rows_v.at[ii % 3], insem).wait()
        pltpu.async_copy(rows_v.at[ii % 3], out_at(ii), outsem)
    for _ in range(n - nwait):
        pltpu.make_async_copy(rows_v.at[0], out_at(0), outsem).wait()


def _make_sc_agg():
    def body(p_hbm, src_hbm, dst_hbm, agg_out,
             src_v, dst_v, rows_v, gsem, isem, ssem, acc_sh):
        c = lax.axis_index("c")
        s = lax.axis_index("s")
        base = s * RPT
        _zero_rows(rows_v.at[0])
        _zero_phase(rows_v.at[0], acc_sh, base, ssem)
        plsc.subcore_barrier()

        pltpu.sync_copy(src_hbm.at[c, s, 0], src_v.at[0])
        pltpu.sync_copy(dst_hbm.at[c, s, 0], dst_v.at[0])
        pltpu.async_copy(p_hbm.at[src_v.at[0, 0]], rows_v.at[0], gsem)

        def wait_one(ref_slices, sem):
            pltpu.make_async_copy(*ref_slices, sem).wait()

        def group(g, carry):
            b = lax.rem(g, 2)
            nb = lax.rem(g + 1, 2)

            @pl.when(g + 1 < NG)
            def _():
                pltpu.async_copy(src_hbm.at[c, s, g + 1], src_v.at[nb],
                                 isem)
                pltpu.async_copy(dst_hbm.at[c, s, g + 1], dst_v.at[nb],
                                 isem)

            for jj in range(G):
                t = g * G + jj
                bt = lax.rem(t, 3)
                bt1 = lax.rem(t + 1, 3)

                @pl.when(t >= 2)
                def _():
                    wait_one((p_hbm.at[src_v.at[b, 0]], rows_v.at[0]),
                             ssem)

                if jj + 1 < G:
                    pltpu.async_copy(p_hbm.at[src_v.at[b, jj + 1]],
                                     rows_v.at[bt1], gsem)
                else:
                    @pl.when(g + 1 < NG)
                    def _():
                        wait_one((src_hbm.at[c, s, 0], src_v.at[nb]),
                                 isem)
                        wait_one((dst_hbm.at[c, s, 0], dst_v.at[nb]),
                                 isem)
                        pltpu.async_copy(p_hbm.at[src_v.at[nb, 0]],
                                         rows_v.at[bt1], gsem)

                wait_one((p_hbm.at[src_v.at[b, jj]], rows_v.at[bt]), gsem)
                pltpu.async_copy(rows_v.at[bt],
                                 acc_sh.at[dst_v.at[b, jj]], ssem,
                                 add=True)
            return carry

        lax.fori_loop(0, NG, group, 0)
        wait_one((p_hbm.at[src_v.at[0, 0]], rows_v.at[0]), ssem)
        wait_one((p_hbm.at[src_v.at[0, 0]], rows_v.at[0]), ssem)
        plsc.subcore_barrier()

        _drain_phase(acc_sh, agg_out, c, base, rows_v, gsem, ssem)

    return pl.kernel(
        body,
        out_type=jax.ShapeDtypeStruct((NC, NP, DD), jnp.float32),
        mesh=mesh_v,
        scratch_types=[
            pltpu.VMEM((2, G, K), jnp.int32),
            pltpu.VMEM((2, G, K), jnp.int32),
            pltpu.VMEM((3, K, DD), jnp.float32),
            pltpu.SemaphoreType.DMA,
            pltpu.SemaphoreType.DMA,
            pltpu.SemaphoreType.DMA,
            pltpu.VMEM_SHARED((NP, DD), jnp.float32),
        ])


def _make_sc_deg():
    def body(dst_hbm, deg_out, dst_v, rows_v, gsem, ssem, acc_sh):
        c = lax.axis_index("c")
        s = lax.axis_index("s")
        base = s * RPT
        _zero_rows(rows_v.at[0])
        _zero_phase(rows_v.at[0], acc_sh, base, ssem)
        plsc.subcore_barrier()
        _zero_rows(rows_v.at[0], 1.0)

        def group(g, carry):
            pltpu.sync_copy(dst_hbm.at[c, s, g], dst_v)
            for jj in range(G):
                pltpu.sync_copy(rows_v.at[0], acc_sh.at[dst_v.at[jj]],
                                add=True)
            return carry

        lax.fori_loop(0, NG, group, 0)
        plsc.subcore_barrier()
        _drain_phase(acc_sh, deg_out, c, base, rows_v, gsem, ssem)

    return pl.kernel(
        body,
        out_type=jax.ShapeDtypeStruct((NC, NP, DD), jnp.float32),
        mesh=mesh_v,
        scratch_types=[
            pltpu.VMEM((G, K), jnp.int32),
            pltpu.VMEM((3, K, DD), jnp.float32),
            pltpu.SemaphoreType.DMA,
            pltpu.SemaphoreType.DMA,
            pltpu.VMEM_SHARED((NP, DD), jnp.float32),
        ])


_TB = 1000


def _tc1(x, W1l, W1r, b1l):
    def body(x_ref, wl_ref, wr_ref, b_ref, p_ref, r_ref):
        xb = x_ref[...]
        p_ref[...] = _dot(xb, wl_ref[...])
        r_ref[...] = _dot(xb, wr_ref[...]) + b_ref[...]

    return pl.pallas_call(
        body,
        grid=(NN // _TB,),
        in_specs=[
            pl.BlockSpec((_TB, DD), lambda i: (i, 0)),
            pl.BlockSpec((DD, DD), lambda i: (0, 0)),
            pl.BlockSpec((DD, DD), lambda i: (0, 0)),
            pl.BlockSpec((1, DD), lambda i: (0, 0)),
        ],
        out_specs=[pl.BlockSpec((_TB, DD), lambda i: (i, 0))] * 2,
        out_shape=[jax.ShapeDtypeStruct((NN, DD), jnp.float32)] * 2,
    )(x, W1l, W1r, b1l.reshape(1, DD))


def _combine(aggp_ref, degp_ref, r_ref):
    agg = aggp_ref[0] + aggp_ref[1]
    deg = degp_ref[0, :, 0:1] + degp_ref[1, :, 0:1]
    deg = jnp.maximum(deg, 1.0)
    return _elu(agg / deg + r_ref[...])


def _tc2(aggp, degp, R1, W2l, W2r, b2l):
    def body(aggp_ref, degp_ref, r1_ref, wl_ref, wr_ref, b_ref,
             p_ref, r_ref):
        h = _combine(aggp_ref, degp_ref, r1_ref)
        p_ref[...] = _dot(h, wl_ref[...])
        r_ref[...] = _dot(h, wr_ref[...]) + b_ref[...]

    return pl.pallas_call(
        body,
        grid=(NN // _TB,),
        in_specs=[
            pl.BlockSpec((NC, _TB, DD), lambda i: (0, i, 0)),
            pl.BlockSpec((NC, _TB, DD), lambda i: (0, i, 0)),
            pl.BlockSpec((_TB, DD), lambda i: (i, 0)),
            pl.BlockSpec((DD, DD), lambda i: (0, 0)),
            pl.BlockSpec((DD, DD), lambda i: (0, 0)),
            pl.BlockSpec((1, DD), lambda i: (0, 0)),
        ],
        out_specs=[pl.BlockSpec((_TB, DD), lambda i: (i, 0))] * 2,
        out_shape=[jax.ShapeDtypeStruct((NN, DD), jnp.float32)] * 2,
    )(aggp, degp, R1, W2l, W2r, b2l.reshape(1, DD))


def _tc3(aggp, degp, R2, Wm1, bm1, Wm2, bm2):
    def body(aggp_ref, degp_ref, r2_ref, w1_ref, b1_ref, w2_ref, b2_ref,
             o_ref):
        h = _combine(aggp_ref, degp_ref, r2_ref)
        t = jnp.maximum(_dot(h, w1_ref[...]) + b1_ref[...], 0.0)
        o_ref[...] = jnp.maximum(_dot(t, w2_ref[...]) + b2_ref[...], 0.0)

    return pl.pallas_call(
        body,
        grid=(NN // _TB,),
        in_specs=[
            pl.BlockSpec((NC, _TB, DD), lambda i: (0, i, 0)),
            pl.BlockSpec((NC, _TB, DD), lambda i: (0, i, 0)),
            pl.BlockSpec((_TB, DD), lambda i: (i, 0)),
            pl.BlockSpec((DD, DD), lambda i: (0, 0)),
            pl.BlockSpec((1, DD), lambda i: (0, 0)),
            pl.BlockSpec((DD, CC), lambda i: (0, 0)),
            pl.BlockSpec((1, CC), lambda i: (0, 0)),
        ],
        out_specs=pl.BlockSpec((_TB, CC), lambda i: (i, 0)),
        out_shape=jax.ShapeDtypeStruct((NN, CC), jnp.float32),
    )(aggp, degp, R2, Wm1, bm1.reshape(1, DD), Wm2, bm2.reshape(1, CC))


def kernel(x, edge_index, W1l, b1l, W1r, W2l, b2l, W2r, Wm1, bm1, Wm2, bm2):
    src4 = edge_index[0].reshape(NC, NS, NG, G, K)
    dst4 = edge_index[1].reshape(NC, NS, NG, G, K)

    sc_agg = _make_sc_agg()
    sc_deg = _make_sc_deg()

    degp = sc_deg(dst4)
    P1, R1 = _tc1(x, W1l, W1r, b1l)
    aggp1 = sc_agg(P1, src4, dst4)
    P2, R2 = _tc2(aggp1, degp, R1, W2l, W2r, b2l)
    aggp2 = sc_agg(P2, src4, dst4)
    return _tc3(aggp2, degp, R2, Wm1, bm1, Wm2, bm2)


# speedup vs baseline: 10.0591x; 1.0101x over previous
import functools

import jax
import jax.numpy as jnp
from jax import lax
from jax.experimental import pallas as pl
from jax.experimental.pallas import tpu as pltpu
from jax.experimental.pallas import tpu_sc as plsc

NN = 10000
DD = 128
CC = 16
EE = 320000

NC = 2
NS = 16
K = 80
CH = EE // (NC * NS * K)
G = 5
NG = CH // G
DW = 128
NP = 10240
RPT = NP // NS


def _dot(a, b):
    return lax.dot_general(a, b, (((1,), (0,)), ((), ())),
                           precision=lax.Precision.HIGHEST,
                           preferred_element_type=jnp.float32)


def _elu(x):
    return jnp.where(x > 0, x, jnp.exp(jnp.minimum(x, 0.0)) - 1.0)


mesh_v = plsc.VectorSubcoreMesh(core_axis_name="c", subcore_axis_name="s")


def _zero_rows(rows_v, val=0.0, width=DD):
    vv = jnp.full((16,), val, jnp.float32)

    def zrow(i, carry):
        for j in range(width // 16):
            rows_v[i, pl.ds(j * 16, 16)] = vv
        return carry

    lax.fori_loop(0, K, zrow, 0)


def _zero_phase(rows0, acc_sh, base, sem):
    for ii in range(RPT // K):
        pltpu.async_copy(rows0, acc_sh.at[pl.ds(base + ii * K, K)], sem)
    for ii in range(RPT // K):
        pltpu.make_async_copy(rows0,
                              acc_sh.at[pl.ds(base, K)], sem).wait()


def _drain_phase(acc_sh, out_hbm, c, base, rows_v, insem, outsem):
    n = RPT // K

    def acc_at(ii):
        return acc_sh.at[pl.ds(base + ii * K, K)]

    def out_at(ii):
        return out_hbm.at[c, pl.ds(base + ii * K, K)]

    pltpu.async_copy(acc_at(0), rows_v.at[0], insem)
    nwait = 0
    for ii in range(n):
        if ii + 1 < n:
            if ii >= 2:
                pltpu.make_async_copy(rows_v.at[0], out_at(0),
                                      outsem).wait()
                nwait += 1
            pltpu.async_copy(acc_at(ii + 1), rows_v.at[(ii + 1) % 3],
                             insem)
        pltpu.make_async_copy(acc_at(ii), rows_v.at[ii % 3], insem).wait()
        pltpu.async_copy(rows_v.at[ii % 3], out_at(ii), outsem)
    for _ in range(n - nwait):
        pltpu.make_async_copy(rows_v.at[0], out_at(0), outsem).wait()


def _make_sc_agg(with_deg):
    def body(*args):
        if with_deg:
            (p_hbm, src_hbm, dst_hbm, agg_out, deg_out,
             src_v, dst_v, rows_v, gsem, isem, ssem, acc_sh) = args
        else:
            (p_hbm, src_hbm, dst_hbm, agg_out,
             src_v, dst_v, rows_v, gsem, isem, ssem, acc_sh) = args
        c = lax.axis_index("c")
        s = lax.axis_index("s")
        base = s * RPT
        _zero_rows(rows_v.at[0])
        _zero_phase(rows_v.at[0], acc_sh, base, ssem)
        plsc.subcore_barrier()

        pltpu.sync_copy(src_hbm.at[c, s, 0], src_v.at[0])
        pltpu.sync_copy(dst_hbm.at[c, s, 0], dst_v.at[0])
        pltpu.async_copy(p_hbm.at[src_v.at[0, 0]], rows_v.at[0], gsem)

        def wait_one(ref_slices, sem):
            pltpu.make_async_copy(*ref_slices, sem).wait()

        def group(g, carry):
            b = lax.rem(g, 2)
            nb = lax.rem(g + 1, 2)

            @pl.when(g + 1 < NG)
            def _():
                pltpu.async_copy(src_hbm.at[c, s, g + 1], src_v.at[nb],
                                 isem)
                pltpu.async_copy(dst_hbm.at[c, s, g + 1], dst_v.at[nb],
                                 isem)

            for jj in range(G):
                t = g * G + jj
                bt = lax.rem(t, 3)
                bt1 = lax.rem(t + 1, 3)

                @pl.when(t >= 2)
                def _():
                    wait_one((p_hbm.at[src_v.at[b, 0]], rows_v.at[0]),
                             ssem)

                if jj + 1 < G:
                    pltpu.async_copy(p_hbm.at[src_v.at[b, jj + 1]],
                                     rows_v.at[bt1], gsem)
                else:
                    @pl.when(g + 1 < NG)
                    def _():
                        wait_one((src_hbm.at[c, s, 0], src_v.at[nb]),
                                 isem)
                        wait_one((dst_hbm.at[c, s, 0], dst_v.at[nb]),
                                 isem)
                        pltpu.async_copy(p_hbm.at[src_v.at[nb, 0]],
                                         rows_v.at[bt1], gsem)

                wait_one((p_hbm.at[src_v.at[b, jj]], rows_v.at[bt]), gsem)
                pltpu.async_copy(rows_v.at[bt],
                                 acc_sh.at[dst_v.at[b, jj]], ssem,
                                 add=True)
            return carry

        lax.fori_loop(0, NG, group, 0)
        wait_one((p_hbm.at[src_v.at[0, 0]], rows_v.at[0]), ssem)
        wait_one((p_hbm.at[src_v.at[0, 0]], rows_v.at[0]), ssem)
        plsc.subcore_barrier()

        _drain_phase(acc_sh, agg_out, c, base, rows_v, gsem, ssem)

        if with_deg:
            _zero_rows(rows_v.at[0])
            _zero_phase(rows_v.at[0], acc_sh, base, ssem)
            plsc.subcore_barrier()
            _zero_rows(rows_v.at[0], 1.0)

            def dgroup(g, carry):
                pltpu.sync_copy(dst_hbm.at[c, s, g], dst_v.at[0])
                for jj in range(G):
                    pltpu.sync_copy(rows_v.at[0],
                                    acc_sh.at[dst_v.at[0, jj]], add=True)
                return carry

            lax.fori_loop(0, NG, dgroup, 0)
            plsc.subcore_barrier()
            _drain_phase(acc_sh, deg_out, c, base, rows_v, gsem, ssem)

    out_types = [jax.ShapeDtypeStruct((NC, NP, DD), jnp.float32)]
    if with_deg:
        out_types.append(jax.ShapeDtypeStruct((NC, NP, DW), jnp.float32))
    return pl.kernel(
        body,
        out_type=out_types if with_deg else out_types[0],
        mesh=mesh_v,
        scratch_types=[
            pltpu.VMEM((2, G, K), jnp.int32),
            pltpu.VMEM((2, G, K), jnp.int32),
            pltpu.VMEM((3, K, DD), jnp.float32),
            pltpu.SemaphoreType.DMA,
            pltpu.SemaphoreType.DMA,
            pltpu.SemaphoreType.DMA,
            pltpu.VMEM_SHARED((NP, DD), jnp.float32),
        ])


_TB = 1000


def _tc1(x, W1l, W1r, b1l):
    def body(x_ref, wl_ref, wr_ref, b_ref, p_ref, r_ref):
        xb = x_ref[...]
        p_ref[...] = _dot(xb, wl_ref[...])
        r_ref[...] = _dot(xb, wr_ref[...]) + b_ref[...]

    return pl.pallas_call(
        body,
        grid=(NN // _TB,),
        in_specs=[
            pl.BlockSpec((_TB, DD), lambda i: (i, 0)),
            pl.BlockSpec((DD, DD), lambda i: (0, 0)),
            pl.BlockSpec((DD, DD), lambda i: (0, 0)),
            pl.BlockSpec((1, DD), lambda i: (0, 0)),
        ],
        out_specs=[pl.BlockSpec((_TB, DD), lambda i: (i, 0))] * 2,
        out_shape=[jax.ShapeDtypeStruct((NN, DD), jnp.float32)] * 2,
    )(x, W1l, W1r, b1l.reshape(1, DD))


def _combine(aggp_ref, degp_ref, r_ref):
    agg = aggp_ref[0] + aggp_ref[1]
    deg = degp_ref[0, :, 0:1] + degp_ref[1, :, 0:1]
    deg = jnp.maximum(deg, 1.0)
    return _elu(agg / deg + r_ref[...])


def _tc2(aggp, degp, R1, W2l, W2r, b2l):
    def body(aggp_ref, degp_ref, r1_ref, wl_ref, wr_ref, b_ref,
             p_ref, r_ref):
        h = _combine(aggp_ref, degp_ref, r1_ref)
        p_ref[...] = _dot(h, wl_ref[...])
        r_ref[...] = _dot(h, wr_ref[...]) + b_ref[...]

    return pl.pallas_call(
        body,
        grid=(NN // _TB,),
        in_specs=[
            pl.BlockSpec((NC, _TB, DD), lambda i: (0, i, 0)),
            pl.BlockSpec((NC, _TB, DW), lambda i: (0, i, 0)),
            pl.BlockSpec((_TB, DD), lambda i: (i, 0)),
            pl.BlockSpec((DD, DD), lambda i: (0, 0)),
            pl.BlockSpec((DD, DD), lambda i: (0, 0)),
            pl.BlockSpec((1, DD), lambda i: (0, 0)),
        ],
        out_specs=[pl.BlockSpec((_TB, DD), lambda i: (i, 0))] * 2,
        out_shape=[jax.ShapeDtypeStruct((NN, DD), jnp.float32)] * 2,
    )(aggp, degp, R1, W2l, W2r, b2l.reshape(1, DD))


def _tc3(aggp, degp, R2, Wm1, bm1, Wm2, bm2):
    def body(aggp_ref, degp_ref, r2_ref, w1_ref, b1_ref, w2_ref, b2_ref,
             o_ref):
        h = _combine(aggp_ref, degp_ref, r2_ref)
        t = jnp.maximum(_dot(h, w1_ref[...]) + b1_ref[...], 0.0)
        o_ref[...] = jnp.maximum(_dot(t, w2_ref[...]) + b2_ref[...], 0.0)

    return pl.pallas_call(
        body,
        grid=(NN // _TB,),
        in_specs=[
            pl.BlockSpec((NC, _TB, DD), lambda i: (0, i, 0)),
            pl.BlockSpec((NC, _TB, DW), lambda i: (0, i, 0)),
            pl.BlockSpec((_TB, DD), lambda i: (i, 0)),
            pl.BlockSpec((DD, DD), lambda i: (0, 0)),
            pl.BlockSpec((1, DD), lambda i: (0, 0)),
            pl.BlockSpec((DD, CC), lambda i: (0, 0)),
            pl.BlockSpec((1, CC), lambda i: (0, 0)),
        ],
        out_specs=pl.BlockSpec((_TB, CC), lambda i: (i, 0)),
        out_shape=jax.ShapeDtypeStruct((NN, CC), jnp.float32),
    )(aggp, degp, R2, Wm1, bm1.reshape(1, DD), Wm2, bm2.reshape(1, CC))


def kernel(x, edge_index, W1l, b1l, W1r, W2l, b2l, W2r, Wm1, bm1, Wm2, bm2):
    src4 = edge_index[0].reshape(NC, NS, NG, G, K)
    dst4 = edge_index[1].reshape(NC, NS, NG, G, K)

    sc_agg_deg = _make_sc_agg(with_deg=True)
    sc_agg = _make_sc_agg(with_deg=False)

    P1, R1 = _tc1(x, W1l, W1r, b1l)
    aggp1, degp = sc_agg_deg(P1, src4, dst4)
    P2, R2 = _tc2(aggp1, degp, R1, W2l, W2r, b2l)
    aggp2 = sc_agg(P2, src4, dst4)
    return _tc3(aggp2, degp, R2, Wm1, bm1, Wm2, bm2)
